# baseline (speedup 1.0000x reference)
import functools

import jax
import jax.numpy as jnp
from jax import lax
from jax.experimental import pallas as pl
from jax.experimental.pallas import tpu as pltpu
from jax.experimental.pallas import tpu_sc as plsc

NC = 2
NS = 16
NW = NC * NS
LANES = 16
CH = 128
LOG2_CH = 7
INT_MAX = jnp.iinfo(jnp.int32).max


def _make_kernel(N, D, B):
  ngrp = D // LANES
  mesh = plsc.VectorSubcoreMesh(
      core_axis_name="c", subcore_axis_name="s", num_cores=NC,
      num_subcores=NS)

  @functools.partial(
      pl.kernel,
      out_type=[
          jax.ShapeDtypeStruct((B, D), jnp.float32),
          jax.ShapeDtypeStruct((B, D), jnp.int32),
      ],
      mesh=mesh,
      compiler_params=pltpu.CompilerParams(use_tc_tiling_on_sc=False),
      scratch_types=[
          pltpu.VMEM((CH, D), jnp.float32),
          pltpu.VMEM((CH, D), jnp.float32),
          pltpu.VMEM((B + 24,), jnp.int32),
          pltpu.VMEM((NW + 24,), jnp.int32),
          pltpu.VMEM((1, D), jnp.float32),
          pltpu.VMEM((1, D), jnp.int32),
          pltpu.SemaphoreType.DMA,
          pltpu.SemaphoreType.DMA,
      ],
  )
  def seg_pool(x_hbm, off_hbm, bnd_hbm, out_hbm, attn_hbm,
               buf0, buf1, offv, bndv, ostage, istage, sem0, sem1):
    wid = lax.axis_index("s") * NC + lax.axis_index("c")
    pltpu.sync_copy(off_hbm, offv)
    pltpu.sync_copy(bnd_hbm, bndv)

    def sload(ref, i):
      return ref[pl.ds(i, LANES)][0]

    seg_lo = sload(bndv, wid)
    seg_hi = sload(bndv, wid + 1)

    def seg_body(s, _):
      pair = offv[pl.ds(s, LANES)]
      off = pair[0]
      nxt = pair[1]
      ln = nxt - off
      accs = [jnp.full((LANES,), -jnp.inf, jnp.float32) for _ in range(ngrp)]
      idxs = [jnp.full((LANES,), INT_MAX, jnp.int32) for _ in range(ngrp)]
      nwin = (ln + (CH - 1)) >> LOG2_CH
      nwin_pad = (nwin + 1) & ~1

      def start_of(j):
        return jnp.minimum(off + j * CH, N - CH)

      def issue(j, buf, sem):
        pltpu.async_copy(x_hbm.at[pl.ds(start_of(j), CH)], buf, sem)

      def compute(j, buf, carry):
        accs, idxs = carry
        start = start_of(j)
        r_lo = (off + j * CH) - start
        r_hi = jnp.minimum(nxt - start, CH)

        def row_body(r, carry):
          accs, idxs = carry
          pos = jnp.full((LANES,), start + r - off, jnp.int32)
          naccs = []
          nidxs = []
          for k in range(ngrp):
            row = buf[r, pl.ds(k * LANES, LANES)]
            upd = row > accs[k]
            nidxs.append(jnp.where(upd, pos, idxs[k]))
            naccs.append(jnp.where(upd, row, accs[k]))
          return naccs, nidxs

        return lax.fori_loop(r_lo, r_hi, row_body, (accs, idxs))

      @pl.when(nwin_pad > 0)
      def _():
        issue(0, buf0, sem0)

      def pair_body(jp, carry):
        j0 = 2 * jp

        @pl.when(j0 + 1 < nwin_pad)
        def _():
          issue(j0 + 1, buf1, sem1)

        pltpu.make_async_copy(x_hbm.at[pl.ds(0, CH)], buf0, sem0).wait()
        carry = compute(j0, buf0, carry)
        j1 = j0 + 1

        @pl.when(j1 + 1 < nwin_pad)
        def _():
          issue(j1 + 1, buf0, sem0)

        pltpu.make_async_copy(x_hbm.at[pl.ds(0, CH)], buf1, sem1).wait()
        return compute(j1, buf1, carry)

      accs, idxs = lax.fori_loop(0, nwin_pad >> 1, pair_body, (accs, idxs))
      for k in range(ngrp):
        ostage[0, pl.ds(k * LANES, LANES)] = accs[k]
        istage[0, pl.ds(k * LANES, LANES)] = idxs[k]
      pltpu.sync_copy(ostage, out_hbm.at[pl.ds(s, 1)])
      pltpu.sync_copy(istage, attn_hbm.at[pl.ds(s, 1)])
      return 0

    lax.fori_loop(seg_lo, seg_hi, seg_body, 0)

  return seg_pool


@jax.jit
def kernel(x, lengths):
  N, D = x.shape
  B = lengths.shape[0]
  csum = jnp.cumsum(lengths, dtype=jnp.int32)
  offsets = jnp.zeros((B + 24,), jnp.int32).at[1:B + 1].set(csum)
  targets = (jnp.arange(1, NW, dtype=jnp.int32) * (N // NW)).astype(jnp.int32)
  inner = jnp.searchsorted(csum, targets, side="left").astype(jnp.int32)
  bounds = jnp.zeros((NW + 24,), jnp.int32)
  bounds = bounds.at[1:NW].set(inner).at[NW].set(B)
  out, attn = _make_kernel(N, D, B)(x, offsets, bounds)
  return (out, attn)


# speedup vs baseline: 1.3113x; 1.3113x over previous
import functools

import jax
import jax.numpy as jnp
from jax import lax
from jax.experimental import pallas as pl
from jax.experimental.pallas import tpu as pltpu
from jax.experimental.pallas import tpu_sc as plsc

NC = 2
NS = 16
NW = NC * NS
LANES = 16
CH = 64
LOG2_CH = 6
INT_MAX = jnp.iinfo(jnp.int32).max


def _make_kernel(N, D, B):
  ngrp = D // LANES
  mesh = plsc.VectorSubcoreMesh(
      core_axis_name="c", subcore_axis_name="s", num_cores=NC,
      num_subcores=NS)

  @functools.partial(
      pl.kernel,
      out_type=[
          jax.ShapeDtypeStruct((B, D), jnp.float32),
          jax.ShapeDtypeStruct((B, D), jnp.int32),
      ],
      mesh=mesh,
      compiler_params=pltpu.CompilerParams(use_tc_tiling_on_sc=False),
      scratch_types=[
          pltpu.VMEM((CH, D), jnp.float32),
          pltpu.VMEM((CH, D), jnp.float32),
          pltpu.VMEM((B + 24,), jnp.int32),
          pltpu.VMEM((NW + 24,), jnp.int32),
          pltpu.VMEM((1, D), jnp.float32),
          pltpu.VMEM((1, D), jnp.int32),
          pltpu.SemaphoreType.DMA,
          pltpu.SemaphoreType.DMA,
      ],
  )
  def seg_pool(x_hbm, off_hbm, bnd_hbm, out_hbm, attn_hbm,
               buf0, buf1, offv, bndv, ostage, istage, sem0, sem1):
    wid = lax.axis_index("s") * NC + lax.axis_index("c")
    pltpu.sync_copy(off_hbm, offv)
    pltpu.sync_copy(bnd_hbm, bndv)

    def sload(ref, i):
      return ref[pl.ds(i, LANES)][0]

    seg_lo = sload(bndv, wid)
    seg_hi = sload(bndv, wid + 1)

    def seg_body(s, _):
      pair = offv[pl.ds(s, LANES)]
      off = pair[0]
      nxt = pair[1]
      ln = nxt - off
      accs = [jnp.full((LANES,), -jnp.inf, jnp.float32) for _ in range(ngrp)]
      idxs = [jnp.full((LANES,), INT_MAX, jnp.int32) for _ in range(ngrp)]
      nwin = (ln + (CH - 1)) >> LOG2_CH
      nwin_pad = (nwin + 1) & ~1

      def start_of(j):
        return jnp.maximum(jnp.minimum(off + j * CH, nxt - CH), 0)

      def issue(j, buf, sem):
        pltpu.async_copy(x_hbm.at[pl.ds(start_of(j), CH)], buf, sem)

      def compute(j, buf, carry):
        accs, idxs = carry
        start = start_of(j)
        r_lo = jnp.maximum(off - start, 0)
        r_hi = jnp.minimum(nxt - start, CH) - 1

        def row_body(r, carry):
          accs, idxs = carry
          rr = jnp.clip(r, r_lo, r_hi)
          pos = jnp.full((LANES,), start + rr - off, jnp.int32)
          naccs = []
          nidxs = []
          for k in range(ngrp):
            row = buf[rr, pl.ds(k * LANES, LANES)]
            upd = row > accs[k]
            nidxs.append(jnp.where(upd, pos, idxs[k]))
            naccs.append(jnp.where(upd, row, accs[k]))
          return naccs, nidxs

        return lax.fori_loop(0, CH, row_body, (accs, idxs))

      @pl.when(nwin_pad > 0)
      def _():
        issue(0, buf0, sem0)

      def pair_body(jp, carry):
        j0 = 2 * jp

        @pl.when(j0 + 1 < nwin_pad)
        def _():
          issue(j0 + 1, buf1, sem1)

        pltpu.make_async_copy(x_hbm.at[pl.ds(0, CH)], buf0, sem0).wait()
        carry = compute(j0, buf0, carry)
        j1 = j0 + 1

        @pl.when(j1 + 1 < nwin_pad)
        def _():
          issue(j1 + 1, buf0, sem0)

        pltpu.make_async_copy(x_hbm.at[pl.ds(0, CH)], buf1, sem1).wait()
        return compute(j1, buf1, carry)

      accs, idxs = lax.fori_loop(0, nwin_pad >> 1, pair_body, (accs, idxs))
      for k in range(ngrp):
        ostage[0, pl.ds(k * LANES, LANES)] = accs[k]
        istage[0, pl.ds(k * LANES, LANES)] = idxs[k]
      pltpu.sync_copy(ostage, out_hbm.at[pl.ds(s, 1)])
      pltpu.sync_copy(istage, attn_hbm.at[pl.ds(s, 1)])
      return 0

    lax.fori_loop(seg_lo, seg_hi, seg_body, 0)

  return seg_pool


@jax.jit
def kernel(x, lengths):
  N, D = x.shape
  B = lengths.shape[0]
  csum = jnp.cumsum(lengths, dtype=jnp.int32)
  offsets = jnp.zeros((B + 24,), jnp.int32).at[1:B + 1].set(csum)
  targets = (jnp.arange(1, NW, dtype=jnp.int32) * (N // NW)).astype(jnp.int32)
  inner = jnp.searchsorted(csum, targets, side="left").astype(jnp.int32)
  bounds = jnp.zeros((NW + 24,), jnp.int32)
  bounds = bounds.at[1:NW].set(inner).at[NW].set(B)
  out, attn = _make_kernel(N, D, B)(x, offsets, bounds)
  return (out, attn)


# speedup vs baseline: 2.7309x; 2.0827x over previous
import functools

import jax
import jax.numpy as jnp
from jax import lax
from jax.experimental import pallas as pl
from jax.experimental.pallas import tpu as pltpu
from jax.experimental.pallas import tpu_sc as plsc

NC = 2
NS = 16
NW = NC * NS
LANES = 16
CH = 64
LOG2_CH = 6
SEGCAP = 128
SEGCOST = 48
INT_MAX = jnp.iinfo(jnp.int32).max


def _make_kernel(N, D, B):
  ngrp = D // LANES
  mesh = plsc.VectorSubcoreMesh(
      core_axis_name="c", subcore_axis_name="s", num_cores=NC,
      num_subcores=NS)

  @functools.partial(
      pl.kernel,
      out_type=[
          jax.ShapeDtypeStruct((B, D), jnp.float32),
          jax.ShapeDtypeStruct((B, D), jnp.int32),
      ],
      mesh=mesh,
      compiler_params=pltpu.CompilerParams(use_tc_tiling_on_sc=False),
      scratch_types=[
          pltpu.VMEM((CH, D), jnp.float32),
          pltpu.VMEM((CH, D), jnp.float32),
          pltpu.VMEM((B + 24,), jnp.int32),
          pltpu.VMEM((NW + 24,), jnp.int32),
          pltpu.VMEM((SEGCAP, D), jnp.float32),
          pltpu.VMEM((SEGCAP, D), jnp.int32),
          pltpu.SemaphoreType.DMA,
          pltpu.SemaphoreType.DMA,
          pltpu.SemaphoreType.DMA,
          pltpu.SemaphoreType.DMA,
      ],
  )
  def seg_pool(x_hbm, off_hbm, bnd_hbm, out_hbm, attn_hbm,
               buf0, buf1, offv, bndv, ostage, istage,
               sem0, sem1, semo, semi):
    wid = lax.axis_index("s") * NC + lax.axis_index("c")
    pltpu.sync_copy(off_hbm, offv)
    pltpu.sync_copy(bnd_hbm, bndv)

    def sload(ref, i):
      return ref[pl.ds(i, LANES)][0]

    seg_lo = sload(bndv, wid)
    seg_hi = sload(bndv, wid + 1)

    def seg_body(s, _):
      pair = offv[pl.ds(s, LANES)]
      off = pair[0]
      nxt = pair[1]
      ln = nxt - off
      accs = [jnp.full((LANES,), -jnp.inf, jnp.float32) for _ in range(ngrp)]
      idxs = [jnp.full((LANES,), INT_MAX, jnp.int32) for _ in range(ngrp)]
      nwin = (ln + (CH - 1)) >> LOG2_CH

      def start_of(j):
        return jnp.maximum(jnp.minimum(off + j * CH, nxt - CH), 0)

      def issue(j, buf, sem):
        pltpu.async_copy(x_hbm.at[pl.ds(start_of(j), CH)], buf, sem)

      def wait(buf, sem):
        pltpu.make_async_copy(x_hbm.at[pl.ds(0, CH)], buf, sem).wait()

      def compute(j, buf, carry):
        accs, idxs = carry
        start = start_of(j)
        r_lo = jnp.maximum(off - start, 0)
        r_hi = jnp.minimum(nxt - start, CH) - 1

        def row_body(r, carry):
          accs, idxs = carry
          rr = jnp.clip(r, r_lo, r_hi)
          pos = jnp.full((LANES,), start + rr - off, jnp.int32)
          naccs = []
          nidxs = []
          for k in range(ngrp):
            row = buf[rr, pl.ds(k * LANES, LANES)]
            upd = row > accs[k]
            nidxs.append(jnp.where(upd, pos, idxs[k]))
            naccs.append(jnp.where(upd, row, accs[k]))
          return naccs, nidxs

        return lax.fori_loop(0, CH, row_body, (accs, idxs))

      @pl.when(nwin > 0)
      def _():
        issue(0, buf0, sem0)

      def pair_body(jp, carry):
        j0 = 2 * jp

        @pl.when(j0 + 1 < nwin)
        def _():
          issue(j0 + 1, buf1, sem1)

        wait(buf0, sem0)
        carry = compute(j0, buf0, carry)
        j1 = j0 + 1

        @pl.when(j1 + 1 < nwin)
        def _():
          issue(j1 + 1, buf0, sem0)

        wait(buf1, sem1)
        return compute(j1, buf1, carry)

      accs, idxs = lax.fori_loop(0, nwin >> 1, pair_body, (accs, idxs))

      def odd_tail(t, carry):
        wait(buf0, sem0)
        return compute(nwin - 1, buf0, carry)

      accs, idxs = lax.fori_loop(0, nwin & 1, odd_tail, (accs, idxs))

      i = s - seg_lo
      for k in range(ngrp):
        ostage[i, pl.ds(k * LANES, LANES)] = accs[k]
        istage[i, pl.ds(k * LANES, LANES)] = idxs[k]
      pltpu.async_copy(ostage.at[pl.ds(i, 1)], out_hbm.at[pl.ds(s, 1)], semo)
      pltpu.async_copy(istage.at[pl.ds(i, 1)], attn_hbm.at[pl.ds(s, 1)], semi)
      return 0

    lax.fori_loop(seg_lo, seg_hi, seg_body, 0)

    def drain_body(s, _):
      pltpu.make_async_copy(
          ostage.at[pl.ds(0, 1)], out_hbm.at[pl.ds(s, 1)], semo).wait()
      pltpu.make_async_copy(
          istage.at[pl.ds(0, 1)], attn_hbm.at[pl.ds(s, 1)], semi).wait()
      return 0

    lax.fori_loop(seg_lo, seg_hi, drain_body, 0)

  return seg_pool


@jax.jit
def kernel(x, lengths):
  N, D = x.shape
  B = lengths.shape[0]
  lengths = lengths.astype(jnp.int32)
  csum = jnp.cumsum(lengths, dtype=jnp.int32)
  offsets = jnp.zeros((B + 24,), jnp.int32).at[1:B + 1].set(csum)
  pad_rows = ((lengths + (CH - 1)) >> LOG2_CH) << LOG2_CH
  cost = jnp.cumsum(pad_rows + SEGCOST, dtype=jnp.int32)
  total = N + B * SEGCOST
  targets = (jnp.arange(1, NW, dtype=jnp.int32) *
             (cost[B - 1] // NW)).astype(jnp.int32)
  inner = jnp.searchsorted(cost, targets, side="left").astype(jnp.int32)
  w = jnp.arange(1, NW, dtype=jnp.int32)
  inner = jnp.minimum(inner, w * SEGCAP)
  inner = jnp.maximum(inner, B - (NW - w) * SEGCAP)
  bounds = jnp.zeros((NW + 24,), jnp.int32)
  bounds = bounds.at[1:NW].set(inner).at[NW].set(B)
  out, attn = _make_kernel(N, D, B)(x, offsets, bounds)
  return (out, attn)


# speedup vs baseline: 2.9786x; 1.0907x over previous
import functools

import jax
import jax.numpy as jnp
from jax import lax
from jax.experimental import pallas as pl
from jax.experimental.pallas import tpu as pltpu
from jax.experimental.pallas import tpu_sc as plsc

NC = 2
NS = 16
NW = NC * NS
LANES = 16
CH = 64
LOG2_CH = 6
SEGCAP = 128
SEGCOST = 48
INT_MAX = jnp.iinfo(jnp.int32).max


def _make_kernel(N, D, B):
  ngrp = D // LANES
  nchunk = B // LANES
  mesh = plsc.VectorSubcoreMesh(
      core_axis_name="c", subcore_axis_name="s", num_cores=NC,
      num_subcores=NS)

  @functools.partial(
      pl.kernel,
      out_type=[
          jax.ShapeDtypeStruct((B, D), jnp.float32),
          jax.ShapeDtypeStruct((B, D), jnp.int32),
      ],
      mesh=mesh,
      compiler_params=pltpu.CompilerParams(
          use_tc_tiling_on_sc=False, needs_layout_passes=False),
      scratch_types=[
          pltpu.VMEM((CH, D), jnp.float32),
          pltpu.VMEM((CH, D), jnp.float32),
          pltpu.VMEM((B,), jnp.int32),
          pltpu.VMEM((B + 24,), jnp.int32),
          pltpu.VMEM((B + 24,), jnp.int32),
          pltpu.VMEM((SEGCAP, D), jnp.float32),
          pltpu.VMEM((SEGCAP, D), jnp.int32),
          pltpu.SemaphoreType.DMA,
          pltpu.SemaphoreType.DMA,
          pltpu.SemaphoreType.DMA,
          pltpu.SemaphoreType.DMA,
      ],
  )
  def seg_pool(x_hbm, len_hbm, out_hbm, attn_hbm,
               buf0, buf1, lenv, offv, costv, ostage, istage,
               sem0, sem1, semo, semi):
    wid = lax.axis_index("s") * NC + lax.axis_index("c")
    pltpu.sync_copy(len_hbm, lenv)

    offv[pl.ds(0, LANES)] = jnp.zeros((LANES,), jnp.int32)

    def cum_body(c, carry):
      off_c, cost_c = carry
      lnv = lenv[pl.ds(c * LANES, LANES)]
      inc = plsc.cumsum(lnv) + off_c
      offv[pl.ds(c * LANES + 1, LANES)] = inc
      padded = ((lnv + (CH - 1)) >> LOG2_CH) << LOG2_CH
      cinc = plsc.cumsum(padded + SEGCOST) + cost_c
      costv[pl.ds(c * LANES, LANES)] = cinc
      return inc[LANES - 1], cinc[LANES - 1]

    _, total_cost = lax.fori_loop(
        0, nchunk, cum_body, (jnp.int32(0), jnp.int32(0)))
    per = total_cost >> 5

    def bound_for(w):
      target = w * per

      def cnt_body(c, acc):
        cv = costv[pl.ds(c * LANES, LANES)]
        ones = jnp.where(cv < target, jnp.int32(1), jnp.int32(0))
        return acc + jnp.sum(ones)

      cnt = lax.fori_loop(0, nchunk, cnt_body, jnp.int32(0))
      b = jnp.where(w >= NW, jnp.int32(B), cnt)
      b = jnp.minimum(b, w * SEGCAP)
      b = jnp.maximum(b, B - (NW - w) * SEGCAP)
      return jnp.minimum(jnp.maximum(b, 0), jnp.int32(B))

    seg_lo = bound_for(wid)
    seg_hi = bound_for(wid + 1)

    def seg_body(s, _):
      pair = offv[pl.ds(s, LANES)]
      off = pair[0]
      nxt = pair[1]
      ln = nxt - off
      accs = [jnp.full((LANES,), -jnp.inf, jnp.float32) for _ in range(ngrp)]
      idxs = [jnp.full((LANES,), INT_MAX, jnp.int32) for _ in range(ngrp)]
      nwin = (ln + (CH - 1)) >> LOG2_CH

      def start_of(j):
        return jnp.maximum(jnp.minimum(off + j * CH, nxt - CH), 0)

      def issue(j, buf, sem):
        pltpu.async_copy(x_hbm.at[pl.ds(start_of(j), CH)], buf, sem)

      def wait(buf, sem):
        pltpu.make_async_copy(x_hbm.at[pl.ds(0, CH)], buf, sem).wait()

      def compute(j, buf, carry):
        accs, idxs = carry
        start = start_of(j)
        r_lo = jnp.maximum(off - start, 0)
        r_hi = jnp.minimum(nxt - start, CH) - 1

        def row_body(r, carry):
          accs, idxs = carry
          rr = jnp.clip(r, r_lo, r_hi)
          pos = jnp.full((LANES,), start + rr - off, jnp.int32)
          naccs = []
          nidxs = []
          for k in range(ngrp):
            row = buf[rr, pl.ds(k * LANES, LANES)]
            upd = row > accs[k]
            nidxs.append(jnp.where(upd, pos, idxs[k]))
            naccs.append(jnp.where(upd, row, accs[k]))
          return naccs, nidxs

        return lax.fori_loop(0, CH, row_body, (accs, idxs))

      @pl.when(nwin > 0)
      def _():
        issue(0, buf0, sem0)

      def pair_body(jp, carry):
        j0 = 2 * jp

        @pl.when(j0 + 1 < nwin)
        def _():
          issue(j0 + 1, buf1, sem1)

        wait(buf0, sem0)
        carry = compute(j0, buf0, carry)
        j1 = j0 + 1

        @pl.when(j1 + 1 < nwin)
        def _():
          issue(j1 + 1, buf0, sem0)

        wait(buf1, sem1)
        return compute(j1, buf1, carry)

      accs, idxs = lax.fori_loop(0, nwin >> 1, pair_body, (accs, idxs))

      def odd_tail(t, carry):
        wait(buf0, sem0)
        return compute(nwin - 1, buf0, carry)

      accs, idxs = lax.fori_loop(0, nwin & 1, odd_tail, (accs, idxs))

      i = s - seg_lo
      for k in range(ngrp):
        ostage[i, pl.ds(k * LANES, LANES)] = accs[k]
        istage[i, pl.ds(k * LANES, LANES)] = idxs[k]
      pltpu.async_copy(ostage.at[pl.ds(i, 1)], out_hbm.at[pl.ds(s, 1)], semo)
      pltpu.async_copy(istage.at[pl.ds(i, 1)], attn_hbm.at[pl.ds(s, 1)], semi)
      return 0

    lax.fori_loop(seg_lo, seg_hi, seg_body, 0)

    def drain_body(s, _):
      pltpu.make_async_copy(
          ostage.at[pl.ds(0, 1)], out_hbm.at[pl.ds(s, 1)], semo).wait()
      pltpu.make_async_copy(
          istage.at[pl.ds(0, 1)], attn_hbm.at[pl.ds(s, 1)], semi).wait()
      return 0

    lax.fori_loop(seg_lo, seg_hi, drain_body, 0)

  return seg_pool


@jax.jit
def kernel(x, lengths):
  N, D = x.shape
  B = lengths.shape[0]
  out, attn = _make_kernel(N, D, B)(x, lengths.astype(jnp.int32))
  return (out, attn)
